# baseline (speedup 1.0000x reference)
import functools

import jax
import jax.numpy as jnp
from jax import lax
from jax.experimental import pallas as pl
from jax.experimental.pallas import tpu as pltpu
from jax.experimental.pallas import tpu_sc as plsc


def _make_sc_gather(S0, S1, V, D):
    info = plsc.get_sparse_core_info()
    NC, NS = info.num_cores, info.num_subcores
    NW = NC * NS
    assert S0 % NW == 0
    BW = S0 // NW
    NBUF = 4

    mesh = plsc.VectorSubcoreMesh(core_axis_name="c", subcore_axis_name="s")

    @functools.partial(
        pl.kernel,
        mesh=mesh,
        out_type=jax.ShapeDtypeStruct((S1, D, S0), jnp.float32),
        compiler_params=pltpu.CompilerParams(
            use_tc_tiling_on_sc=False, needs_layout_passes=False
        ),
        scratch_types=[
            pltpu.VMEM((S1, BW), jnp.int32),
            pltpu.VMEM((NBUF, BW, D), jnp.float32),
            pltpu.VMEM((NBUF, D, BW), jnp.float32),
        ]
        + [pltpu.SemaphoreType.DMA] * (2 * NBUF),
    )
    def k(xt_hbm, table_hbm, out_hbm, idx_v, rows_v, outt_v, *sems):
        gsem = sems[:NBUF]
        wsem = sems[NBUF:]
        wid = lax.axis_index("s") * NC + lax.axis_index("c")
        b0 = wid * BW
        pltpu.sync_copy(xt_hbm.at[:, pl.ds(b0, BW)], idx_v)

        def gather_start(j, b):
            pltpu.async_copy(
                table_hbm.at[idx_v.at[j, pl.ds(0, BW)]], rows_v.at[b], gsem[b]
            )

        def gather_wait(b):
            pltpu.make_async_copy(
                table_hbm.at[idx_v.at[0, pl.ds(0, BW)]], rows_v.at[b], gsem[b]
            ).wait()

        def write_start(j, b):
            pltpu.async_copy(
                outt_v.at[b], out_hbm.at[j, :, pl.ds(b0, BW)], wsem[b]
            )

        def write_wait(b):
            pltpu.make_async_copy(
                outt_v.at[b], out_hbm.at[0, :, pl.ds(b0, BW)], wsem[b]
            ).wait()

        def transpose(b):
            iota = lax.iota(jnp.int32, 16)
            bsplat = jnp.zeros((16,), jnp.int32) + b

            def fbody(f, carry):
                fsplat = jnp.zeros((16,), jnp.int32) + f
                for g in range(BW // 16):
                    col = g * 16 + iota
                    vals = plsc.load_gather(rows_v, [bsplat, col, fsplat])
                    outt_v[b, f, pl.ds(g * 16, 16)] = vals
                return carry

            lax.fori_loop(0, D, fbody, 0)

        for b in range(NBUF):
            gather_start(b, b)

        def body(grp, carry):
            for b in range(NBUF):
                j = grp * NBUF + b
                gather_wait(b)

                @pl.when(grp > 0)
                def _():
                    write_wait(b)

                transpose(b)
                write_start(j, b)

                @pl.when(j + NBUF < S1)
                def _():
                    gather_start(j + NBUF, b)

            return carry

        lax.fori_loop(0, S1 // NBUF, body, 0)
        for b in range(NBUF):
            write_wait(b)

    return k


def kernel(x, table):
    S0, S1 = x.shape
    V, D = table.shape
    xt = jnp.transpose(x).astype(jnp.int32)
    out_t = _make_sc_gather(S0, S1, V, D)(xt, table)
    return jnp.transpose(out_t, (2, 0, 1))


# speedup vs baseline: 1.3606x; 1.3606x over previous
import functools

import jax
import jax.numpy as jnp
from jax import lax
from jax.experimental import pallas as pl
from jax.experimental.pallas import tpu as pltpu
from jax.experimental.pallas import tpu_sc as plsc


def _make_sc_gather(S0, S1, V, D):
    info = plsc.get_sparse_core_info()
    NC, NS = info.num_cores, info.num_subcores
    NW = NC * NS
    assert S0 % NW == 0
    BW = S0 // NW
    NBUF = 4

    mesh = plsc.VectorSubcoreMesh(core_axis_name="c", subcore_axis_name="s")

    @functools.partial(
        pl.kernel,
        mesh=mesh,
        out_type=jax.ShapeDtypeStruct((S1, D, S0), jnp.float32),
        compiler_params=pltpu.CompilerParams(
            use_tc_tiling_on_sc=False, needs_layout_passes=False
        ),
        scratch_types=[
            pltpu.VMEM((S1, BW), jnp.int32),
            pltpu.VMEM((NBUF, BW, D), jnp.float32),
            pltpu.VMEM((NBUF, D, BW), jnp.float32),
        ]
        + [pltpu.SemaphoreType.DMA] * (2 * NBUF),
    )
    def k(xt_hbm, table_hbm, out_hbm, idx_v, rows_v, outt_v, *sems):
        gsem = sems[:NBUF]
        wsem = sems[NBUF:]
        wid = lax.axis_index("s") * NC + lax.axis_index("c")
        b0 = wid * BW
        pltpu.sync_copy(xt_hbm.at[:, pl.ds(b0, BW)], idx_v)

        def gather_start(j, b):
            pltpu.async_copy(
                table_hbm.at[idx_v.at[j, pl.ds(0, BW)]], rows_v.at[b], gsem[b]
            )

        def gather_wait(b):
            pltpu.make_async_copy(
                table_hbm.at[idx_v.at[0, pl.ds(0, BW)]], rows_v.at[b], gsem[b]
            ).wait()

        def write_start(j, b):
            pltpu.async_copy(
                outt_v.at[b], out_hbm.at[j, :, pl.ds(b0, BW)], wsem[b]
            )

        def write_wait(b):
            pltpu.make_async_copy(
                outt_v.at[b], out_hbm.at[0, :, pl.ds(b0, BW)], wsem[b]
            ).wait()

        def transpose(b):
            iota = lax.iota(jnp.int32, 16)
            bsplat = jnp.zeros((16,), jnp.int32) + b

            @plsc.parallel_loop(0, D, unroll=8)
            def fbody(f):
                fsplat = jnp.zeros((16,), jnp.int32) + f
                for g in range(BW // 16):
                    col = g * 16 + iota
                    vals = plsc.load_gather(rows_v, [bsplat, col, fsplat])
                    outt_v[b, f, pl.ds(g * 16, 16)] = vals

        for b in range(NBUF):
            gather_start(b, b)

        def body(grp, carry):
            for b in range(NBUF):
                j = grp * NBUF + b
                gather_wait(b)

                @pl.when(grp > 0)
                def _():
                    write_wait(b)

                transpose(b)
                write_start(j, b)

                @pl.when(j + NBUF < S1)
                def _():
                    gather_start(j + NBUF, b)

            return carry

        lax.fori_loop(0, S1 // NBUF, body, 0)
        for b in range(NBUF):
            write_wait(b)

    return k


def kernel(x, table):
    S0, S1 = x.shape
    V, D = table.shape
    xt = jnp.transpose(x).astype(jnp.int32)
    out_t = _make_sc_gather(S0, S1, V, D)(xt, table)
    return jnp.transpose(out_t, (2, 0, 1))


# speedup vs baseline: 1.5560x; 1.1436x over previous
import functools

import jax
import jax.numpy as jnp
from jax import lax
from jax.experimental import pallas as pl
from jax.experimental.pallas import tpu as pltpu
from jax.experimental.pallas import tpu_sc as plsc


def _make_sc_gather(S0, S1, V, D):
    info = plsc.get_sparse_core_info()
    NC, NS = info.num_cores, info.num_subcores
    NW = NC * NS
    assert S0 % (NW * 128) == 0
    BW = S0 // NW
    TR = D // 8
    TC_ = S0 // 128
    NBUF = 4

    mesh = plsc.VectorSubcoreMesh(core_axis_name="c", subcore_axis_name="s")

    @functools.partial(
        pl.kernel,
        mesh=mesh,
        out_type=jax.ShapeDtypeStruct((S1, TR, TC_, 8 * 128), jnp.float32),
        compiler_params=pltpu.CompilerParams(
            use_tc_tiling_on_sc=False, needs_layout_passes=False
        ),
        scratch_types=[
            pltpu.VMEM((S1, BW), jnp.int32),
            pltpu.VMEM((NBUF, BW, D), jnp.float32),
            pltpu.VMEM((NBUF, TR, 8 * 128), jnp.float32),
        ]
        + [pltpu.SemaphoreType.DMA] * (2 * NBUF),
    )
    def k(xt_hbm, table_hbm, out_hbm, idx_v, rows_v, outt_v, *sems):
        gsem = sems[:NBUF]
        wsem = sems[NBUF:]
        wid = lax.axis_index("s") * NC + lax.axis_index("c")
        b0 = wid * BW
        pltpu.sync_copy(xt_hbm.at[:, pl.ds(b0, BW)], idx_v)

        def gather_start(j, b):
            pltpu.async_copy(
                table_hbm.at[idx_v.at[j, pl.ds(0, BW)]], rows_v.at[b], gsem[b]
            )

        def gather_wait(b):
            pltpu.make_async_copy(
                table_hbm.at[idx_v.at[0, pl.ds(0, BW)]], rows_v.at[b], gsem[b]
            ).wait()

        def write_start(j, b):
            pltpu.async_copy(
                outt_v.at[b], out_hbm.at[j, :, wid, :], wsem[b]
            )

        def write_wait(b):
            pltpu.make_async_copy(
                outt_v.at[b], out_hbm.at[0, :, wid, :], wsem[b]
            ).wait()

        iota = lax.iota(jnp.int32, 16)
        bvecs = [jnp.zeros((16,), jnp.int32) + b for b in range(NBUF)]
        d1c = [(g * 16 + iota) >> 3 for g in range(D // 16)]
        d2c = [((g * 16 + iota) & 7) * 128 for g in range(D // 16)]

        def transpose(b):
            @plsc.parallel_loop(0, BW, unroll=8)
            def tbody(t):
                tvec = jnp.zeros((16,), jnp.int32) + t
                for g in range(D // 16):
                    vals = rows_v[b, t, pl.ds(g * 16, 16)]
                    plsc.store_scatter(
                        outt_v, [bvecs[b], d1c[g], d2c[g] + tvec], vals
                    )

        for b in range(NBUF):
            gather_start(b, b)

        def body(grp, carry):
            for b in range(NBUF):
                j = grp * NBUF + b
                gather_wait(b)

                @pl.when(grp > 0)
                def _():
                    write_wait(b)

                transpose(b)
                write_start(j, b)

                @pl.when(j + NBUF < S1)
                def _():
                    gather_start(j + NBUF, b)

            return carry

        lax.fori_loop(0, S1 // NBUF, body, 0)
        for b in range(NBUF):
            write_wait(b)

    return k


def kernel(x, table):
    S0, S1 = x.shape
    V, D = table.shape
    xt = jnp.transpose(x).astype(jnp.int32)
    o = _make_sc_gather(S0, S1, V, D)(xt, table)
    o = o.reshape(S1, D // 8, S0 // 128, 8, 128)
    o = o.transpose(0, 1, 3, 2, 4)
    o = o.reshape(S1, D, S0)
    return o.transpose(2, 0, 1)


# speedup vs baseline: 2.7077x; 1.7402x over previous
import functools

import jax
import jax.numpy as jnp
from jax import lax
from jax.experimental import pallas as pl
from jax.experimental.pallas import tpu as pltpu
from jax.experimental.pallas import tpu_sc as plsc


def _make_sc_gather(S0, S1, V, D):
    info = plsc.get_sparse_core_info()
    NC, NS = info.num_cores, info.num_subcores
    NW = NC * NS
    assert S0 % (NW * 128) == 0
    BW = S0 // NW
    TR = D // 8
    TC_ = S0 // 128
    NBUF = 4

    mesh = plsc.VectorSubcoreMesh(core_axis_name="c", subcore_axis_name="s")

    @functools.partial(
        pl.kernel,
        mesh=mesh,
        out_type=jax.ShapeDtypeStruct((S1, TR, TC_, 8, 128), jnp.float32),
        compiler_params=pltpu.CompilerParams(
            use_tc_tiling_on_sc=False, needs_layout_passes=False
        ),
        scratch_types=[
            pltpu.VMEM((S1, BW), jnp.int32),
            pltpu.VMEM((NBUF, BW, D), jnp.float32),
            pltpu.VMEM((NBUF, TR, 8, 137), jnp.float32),
        ]
        + [pltpu.SemaphoreType.DMA] * (2 * NBUF),
    )
    def k(xt_hbm, table_hbm, out_hbm, idx_v, rows_v, outt_v, *sems):
        gsem = sems[:NBUF]
        wsem = sems[NBUF:]
        wid = lax.axis_index("s") * NC + lax.axis_index("c")
        b0 = wid * BW
        pltpu.sync_copy(xt_hbm.at[:, pl.ds(b0, BW)], idx_v)

        def gather_start(j, b):
            pltpu.async_copy(
                table_hbm.at[idx_v.at[j, pl.ds(0, BW)]], rows_v.at[b], gsem[b]
            )

        def gather_wait(b):
            pltpu.make_async_copy(
                table_hbm.at[idx_v.at[0, pl.ds(0, BW)]], rows_v.at[b], gsem[b]
            ).wait()

        def write_start(j, b):
            pltpu.async_copy(
                outt_v.at[b, :, :, pl.ds(0, 128)], out_hbm.at[j, :, wid, :, :],
                wsem[b],
            )

        def write_wait(b):
            pltpu.make_async_copy(
                outt_v.at[b, :, :, pl.ds(0, 128)], out_hbm.at[0, :, wid, :, :],
                wsem[b],
            ).wait()

        iota = lax.iota(jnp.int32, 16)
        bvecs = [jnp.zeros((16,), jnp.int32) + b for b in range(NBUF)]
        d1c = [(g * 16 + iota) >> 3 for g in range(D // 16)]
        d2c = [(g * 16 + iota) & 7 for g in range(D // 16)]

        def transpose(b):
            @plsc.parallel_loop(0, BW, unroll=8)
            def tbody(t):
                tvec = jnp.zeros((16,), jnp.int32) + t
                for g in range(D // 16):
                    vals = rows_v[b, t, pl.ds(g * 16, 16)]
                    plsc.store_scatter(
                        outt_v, [bvecs[b], d1c[g], d2c[g], tvec], vals
                    )

        for b in range(NBUF):
            gather_start(b, b)

        def body(grp, carry):
            for b in range(NBUF):
                j = grp * NBUF + b
                gather_wait(b)

                @pl.when(grp > 0)
                def _():
                    write_wait(b)

                transpose(b)
                write_start(j, b)

                @pl.when(j + NBUF < S1)
                def _():
                    gather_start(j + NBUF, b)

            return carry

        lax.fori_loop(0, S1 // NBUF, body, 0)
        for b in range(NBUF):
            write_wait(b)

    return k


def kernel(x, table):
    S0, S1 = x.shape
    V, D = table.shape
    xt = jnp.transpose(x).astype(jnp.int32)
    o = _make_sc_gather(S0, S1, V, D)(xt, table)
    o = o.transpose(0, 1, 3, 2, 4)
    o = o.reshape(S1, D, S0)
    return o.transpose(2, 0, 1)
